# baseline (speedup 1.0000x reference)
import functools

import jax
import jax.numpy as jnp
from jax import lax
from jax.experimental import pallas as pl
from jax.experimental.pallas import tpu as pltpu
from jax.experimental.pallas import tpu_sc as plsc

N = 100000
C = 10000
D = 128
H = 64

R = 2000
GRID = N // R
NT = 32
CW = 313
CH = 512
NP = N + CH

_F32 = jnp.float32


def _rsqrt_precise(v):
    r = lax.rsqrt(v)
    return r * (1.5 - 0.5 * v * r * r)


def _ln_relu(z, gm, bt):
    mu = jnp.mean(z, axis=-1, keepdims=True)
    zc = z - mu
    var = jnp.mean(zc * zc, axis=-1, keepdims=True)
    h = zc * _rsqrt_precise(var + 1e-5) * gm + bt
    return jnp.maximum(h, 0.0)


def _dot(a, b):
    return lax.dot_general(a, b, (((1,), (0,)), ((), ())),
                           preferred_element_type=_F32)


def _dot_hi(a, b):
    return lax.dot_general(a, b, (((1,), (0,)), ((), ())),
                           preferred_element_type=_F32,
                           precision=lax.Precision.HIGHEST)


def _mlp0_body(x_ref, cat_ref, w_ref, m_ref, b_ref, gm_ref, bt_ref,
               h_ref, s_ref):
    i = pl.program_id(0)
    z = _dot(x_ref[...], w_ref[...])
    zp = jnp.concatenate([z[:R // 2], z[R // 2:]], axis=1) + b_ref[...]
    hp = _ln_relu_pair(zp, m_ref, gm_ref[...], bt_ref[...])
    h_ref[...] = jnp.concatenate([hp[:, :64], hp[:, 64:]], axis=0)
    cat = cat_ref[0]
    th = lax.broadcasted_iota(jnp.int32, (1, 128), 1) * CW
    cmp = (cat < th).astype(_F32)
    ssum = jnp.sum(cmp, axis=0, keepdims=True)

    @pl.when(i == 0)
    def _():
        s_ref[...] = jnp.zeros_like(s_ref)
    s_ref[...] += ssum


def _ln_relu_pair(z, m_ref, gm, bt):
    mu = _dot_hi(z, m_ref[...])
    zc = z - mu
    var = _dot_hi(zc * zc, m_ref[...])
    h = zc * _rsqrt_precise(var + 1e-5) * gm + bt
    return jnp.maximum(h, 0.0)


def _mlp_mid_body(h_ref, g_ref, wt_ref, wb_ref, m_ref, b_ref, gm_ref,
                  bt_ref, o_ref):
    hv = h_ref[...]
    gv = g_ref[...]
    z = _dot(hv, wt_ref[...]) + _dot(gv, wb_ref[...]) + b_ref[...]
    h = _ln_relu_pair(z, m_ref, gm_ref[...], bt_ref[...])
    o_ref[...] = h


def _mlp_last_body(h_ref, g_ref, wt_ref, wb_ref, m_ref, b_ref, gm_ref,
                   bt_ref, o_ref, ss_ref):
    i = pl.program_id(0)
    hv = h_ref[...]
    gv = g_ref[...]
    z = _dot(hv, wt_ref[...]) + _dot(gv, wb_ref[...]) + b_ref[...]
    h = _ln_relu_pair(z, m_ref, gm_ref[...], bt_ref[...])
    o_ref[...] = h

    @pl.when(i == 0)
    def _():
        ss_ref[...] = jnp.zeros_like(ss_ref)
    ss_ref[...] += jnp.sum(h * h, axis=0, keepdims=True)


def _colsq_body(g_ref, ss_ref):
    i = pl.program_id(0)

    @pl.when(i == 0)
    def _():
        ss_ref[...] = jnp.zeros_like(ss_ref)
    g = g_ref[...]
    ss_ref[...] += jnp.sum(g * g, axis=0, keepdims=True)


def _final_body(h_ref, g_ref, ssh_ref, ssg_ref, y_ref):
    ssh_p = ssh_ref[...]
    ssg_p = ssg_ref[...]
    ih = _rsqrt_precise(ssh_p[:, :64] + ssh_p[:, 64:] + 1e-30)
    ig = _rsqrt_precise(ssg_p[:, :64] + ssg_p[:, 64:] + 1e-30)
    hp = h_ref[...]
    gp = g_ref[...]
    yp = jnp.concatenate(
        [hp[:, :64] * ih, gp[:, :64] * ig, hp[:, 64:] * ih, gp[:, 64:] * ig],
        axis=1)
    y_ref[...] = yp.reshape(R, 128)


NP64 = NP * 64
NPH = NP // 2


def _flat_spec():
    return pl.BlockSpec((R // 2, 128), lambda i: (i, 0))


def _const_spec(shape):
    nd = len(shape)
    return pl.BlockSpec(shape, lambda i: (0,) * nd)


def _tc_mlp0(x, cat3, w, m, b, gm, bt):
    return pl.pallas_call(
        _mlp0_body,
        grid=(GRID,),
        in_specs=[
            pl.BlockSpec((R, 128), lambda i: (i, 0)),
            pl.BlockSpec((1, R, 1), lambda i: (i, 0, 0)),
            _const_spec((128, 64)),
            _const_spec((128, 128)),
            _const_spec((1, 128)),
            _const_spec((1, 128)),
            _const_spec((1, 128)),
        ],
        out_specs=[pl.BlockSpec((R, 64), lambda i: (i, 0)),
                   _const_spec((1, 128))],
        out_shape=[
            jax.ShapeDtypeStruct((NP, 64), _F32),
            jax.ShapeDtypeStruct((1, 128), _F32),
        ],
    )(x, cat3, w, m, b, gm, bt)


def _tc_mid(h, g, wt, wb, m, b, gm, bt):
    return pl.pallas_call(
        _mlp_mid_body,
        grid=(GRID,),
        in_specs=[
            _flat_spec(), _flat_spec(),
            _const_spec((128, 128)), _const_spec((128, 128)),
            _const_spec((128, 128)),
            _const_spec((1, 128)), _const_spec((1, 128)),
            _const_spec((1, 128)),
        ],
        out_specs=_flat_spec(),
        out_shape=jax.ShapeDtypeStruct((NPH, 128), _F32),
    )(h, g, wt, wb, m, b, gm, bt)


def _tc_last(h, g, wt, wb, m, b, gm, bt):
    return pl.pallas_call(
        _mlp_last_body,
        grid=(GRID,),
        in_specs=[
            _flat_spec(), _flat_spec(),
            _const_spec((128, 128)), _const_spec((128, 128)),
            _const_spec((128, 128)),
            _const_spec((1, 128)), _const_spec((1, 128)),
            _const_spec((1, 128)),
        ],
        out_specs=[_flat_spec(), _const_spec((1, 128))],
        out_shape=[
            jax.ShapeDtypeStruct((NPH, 128), _F32),
            jax.ShapeDtypeStruct((1, 128), _F32),
        ],
    )(h, g, wt, wb, m, b, gm, bt)


def _tc_colsq(g):
    return pl.pallas_call(
        _colsq_body,
        grid=(GRID,),
        in_specs=[_flat_spec()],
        out_specs=_const_spec((1, 128)),
        out_shape=jax.ShapeDtypeStruct((1, 128), _F32),
    )(g)


def _tc_final(h, g, ssh, ssg):
    return pl.pallas_call(
        _final_body,
        grid=(GRID,),
        in_specs=[
            _flat_spec(), _flat_spec(),
            _const_spec((1, 128)), _const_spec((1, 128)),
        ],
        out_specs=pl.BlockSpec((R, 128), lambda i: (i, 0)),
        out_shape=jax.ShapeDtypeStruct((N, 128), _F32),
    )(h, g, ssh, ssg)


def _sc_mesh():
    return plsc.VectorSubcoreMesh(core_axis_name="c", subcore_axis_name="s",
                                  num_cores=2, num_subcores=16)


def _segmax_gather_body(h_hbm, cat_hbm, st_hbm, g_hbm,
                        sv, cb0, cb1, hb0, hb1, abuf, gout,
                        sem_c0, sem_c1, sem_h0, sem_h1):
    wid = lax.axis_index("s") * 2 + lax.axis_index("c")
    pltpu.sync_copy(st_hbm.at[pl.ds(0, 48)], sv.at[pl.ds(0, 48)])
    sw = sv[pl.ds(wid, 16)]
    start = sw[0]
    end = sw[1]
    c_lo = wid * CW
    c_hi = c_lo + CW

    zv = jnp.zeros((16,), _F32)

    def zb(i, _):
        abuf[pl.ds(i * 16, 16)] = zv
        return 0
    lax.fori_loop(0, (CW + 1) * 4, zb, 0)

    a0 = (start // 8) * 8
    total = end - a0
    nch = (total + CH - 1) // CH
    maxr0 = NP - CH

    def c_off(k):
        return jnp.minimum(a0 + k * CH, maxr0)

    def start_dma(k, cb, hb, sem_c, sem_h):
        r0 = c_off(k)
        pltpu.async_copy(cat_hbm.at[pl.ds(r0, CH)], cb, sem_c)
        pltpu.async_copy(h_hbm.at[pl.ds(r0 * 64, CH * 64)], hb, sem_h)

    def wait_dma(cb, hb, sem_c, sem_h):
        pltpu.make_async_copy(cat_hbm.at[pl.ds(0, CH)], cb, sem_c).wait()
        pltpu.make_async_copy(h_hbm.at[pl.ds(0, CH * 64)], hb, sem_h).wait()

    def scan_chunk(cb, hb, carry):
        def blk(bi, c2):
            prev, q0, q1, q2, q3 = c2
            catv = cb[pl.ds(bi * 16, 16)]
            for j in range(16):
                c = catv[j]
                keep = jnp.where(c != prev, 0.0, 1.0).astype(_F32)
                base = bi * 1024 + j * 64
                q0 = jnp.maximum(hb[pl.ds(base, 16)], q0 * keep)
                q1 = jnp.maximum(hb[pl.ds(base + 16, 16)], q1 * keep)
                q2 = jnp.maximum(hb[pl.ds(base + 32, 16)], q2 * keep)
                q3 = jnp.maximum(hb[pl.ds(base + 48, 16)], q3 * keep)
                lc = jnp.where((c >= c_lo) & (c < c_hi), c - c_lo, CW)
                ab = lc * 64
                abuf[pl.ds(ab, 16)] = q0
                abuf[pl.ds(ab + 16, 16)] = q1
                abuf[pl.ds(ab + 32, 16)] = q2
                abuf[pl.ds(ab + 48, 16)] = q3
                prev = c
            return (prev, q0, q1, q2, q3)
        return lax.fori_loop(0, CH // 16, blk, carry)

    start_dma(0, cb0, hb0, sem_c0, sem_h0)
    npair = jnp.maximum((nch + 1) // 2, 1)

    def pair(i, carry):
        k0 = 2 * i
        wait_dma(cb0, hb0, sem_c0, sem_h0)
        start_dma(k0 + 1, cb1, hb1, sem_c1, sem_h1)
        carry = scan_chunk(cb0, hb0, carry)
        wait_dma(cb1, hb1, sem_c1, sem_h1)
        start_dma(k0 + 2, cb0, hb0, sem_c0, sem_h0)
        carry = scan_chunk(cb1, hb1, carry)
        return carry

    lax.fori_loop(0, npair, pair, (jnp.int32(-1), zv, zv, zv, zv))
    wait_dma(cb0, hb0, sem_c0, sem_h0)

    def chunk2(k, _):
        r0 = c_off(k)
        pltpu.sync_copy(cat_hbm.at[pl.ds(r0, CH)], cb0)
        lo = jnp.maximum(start - r0, 0)
        hi = jnp.minimum(end - r0, CH)

        def blk(bi, _):
            catv = cb0[pl.ds(bi * 16, 16)]
            for j in range(16):
                c = catv[j]
                lc = jnp.where((c >= c_lo) & (c < c_hi), c - c_lo, CW)
                ab = lc * 64
                rb = bi * 1024 + j * 64
                gout[pl.ds(rb, 16)] = abuf[pl.ds(ab, 16)]
                gout[pl.ds(rb + 16, 16)] = abuf[pl.ds(ab + 16, 16)]
                gout[pl.ds(rb + 32, 16)] = abuf[pl.ds(ab + 32, 16)]
                gout[pl.ds(rb + 48, 16)] = abuf[pl.ds(ab + 48, 16)]
            return 0

        lax.fori_loop(0, CH // 16, blk, 0)

        ln = jnp.maximum(hi - lo, 0)
        for b in range(9, -1, -1):
            blen = 1 << b
            rowstart = lo + ((ln >> (b + 1)) << (b + 1))

            @pl.when((ln & blen) != 0)
            def _(rowstart=rowstart, blen=blen):
                pltpu.sync_copy(
                    gout.at[pl.ds(rowstart * 64, blen * 64)],
                    g_hbm.at[pl.ds((r0 + rowstart) * 64, blen * 64)])
        return 0

    lax.fori_loop(0, nch, chunk2, 0)


def _sc_segmax_gather(h_flat, cat, starts):
    run = functools.partial(
        pl.kernel,
        out_type=jax.ShapeDtypeStruct((NP * 64,), _F32),
        mesh=_sc_mesh(),
        scratch_types=[
            pltpu.VMEM((64,), jnp.int32),
            pltpu.VMEM((CH,), jnp.int32),
            pltpu.VMEM((CH,), jnp.int32),
            pltpu.VMEM((CH * 64,), _F32),
            pltpu.VMEM((CH * 64,), _F32),
            pltpu.VMEM(((CW + 1) * 64,), _F32),
            pltpu.VMEM((CH * 64,), _F32),
            pltpu.SemaphoreType.DMA,
            pltpu.SemaphoreType.DMA,
            pltpu.SemaphoreType.DMA,
            pltpu.SemaphoreType.DMA,
        ],
    )(_segmax_gather_body)
    return run(h_flat, cat, starts)


def _bd2(w):
    z = jnp.zeros((128, 128), _F32)
    return z.at[:64, :64].set(w).at[64:, 64:].set(w)


def kernel(x, category, W0, b0, g0, beta0, W1, b1, g1, beta1,
           W2, b2, g2, beta2):
    cat = category.astype(jnp.int32)
    cat_pad = jnp.pad(cat, (0, NP - N))
    cat3 = cat.reshape(GRID, R, 1)

    def r2(v):
        return v.reshape(1, 64)

    def p2(v):
        return jnp.tile(v, 2).reshape(1, 128)

    m_avg = _bd2(jnp.full((64, 64), 1.0 / 64.0, _F32))

    h0, starts_f = _tc_mlp0(x, cat3, W0, m_avg, p2(b0), p2(g0), p2(beta0))
    starts = starts_f.astype(jnp.int32).reshape(128)

    def fl(v):
        return v.reshape(NP64)

    def pr(v):
        return v.reshape(NPH, 128)

    h0f = h0.reshape(NP64)
    g0v = pr(_sc_segmax_gather(h0f, cat_pad, starts))
    h1 = _tc_mid(pr(h0f), g0v, _bd2(W1[:64]), _bd2(W1[64:]), m_avg,
                 p2(b1), p2(g1), p2(beta1))

    g1v = pr(_sc_segmax_gather(fl(h1), cat_pad, starts))
    h2, ssh = _tc_last(h1, g1v, _bd2(W2[:64]), _bd2(W2[64:]), m_avg,
                       p2(b2), p2(g2), p2(beta2))

    ga2 = _sc_segmax_gather(fl(h2), cat_pad, starts)
    ssg = _tc_colsq(pr(ga2))
    return _tc_final(h2, pr(ga2), ssh, ssg)


# speedup vs baseline: 1.1104x; 1.1104x over previous
import functools

import jax
import jax.numpy as jnp
from jax import lax
from jax.experimental import pallas as pl
from jax.experimental.pallas import tpu as pltpu
from jax.experimental.pallas import tpu_sc as plsc

N = 100000
C = 10000
D = 128
H = 64

R = 2000
GRID = N // R
NT = 32
CW = 313
CH = 512
NP = N + CH

_F32 = jnp.float32


def _rsqrt_precise(v):
    r = lax.rsqrt(v)
    return r * (1.5 - 0.5 * v * r * r)


def _ln_relu(z, gm, bt):
    mu = jnp.mean(z, axis=-1, keepdims=True)
    zc = z - mu
    var = jnp.mean(zc * zc, axis=-1, keepdims=True)
    h = zc * _rsqrt_precise(var + 1e-5) * gm + bt
    return jnp.maximum(h, 0.0)


def _dot(a, b):
    return lax.dot_general(a, b, (((1,), (0,)), ((), ())),
                           preferred_element_type=_F32)


def _dot_hi(a, b):
    return lax.dot_general(a, b, (((1,), (0,)), ((), ())),
                           preferred_element_type=_F32,
                           precision=lax.Precision.HIGHEST)


def _mlp0_body(x_ref, cat_ref, w_ref, b_ref, gm_ref, bt_ref, h_ref, s_ref):
    i = pl.program_id(0)
    z = _dot(x_ref[...], w_ref[...]) + b_ref[...]
    h = _ln_relu(z, gm_ref[...], bt_ref[...])
    h_ref[...] = h
    cat = cat_ref[0]
    th = lax.broadcasted_iota(jnp.int32, (1, 128), 1) * CW
    cmp = (cat < th).astype(_F32)
    ssum = jnp.sum(cmp, axis=0, keepdims=True)

    @pl.when(i == 0)
    def _():
        s_ref[...] = jnp.zeros_like(s_ref)
    s_ref[...] += ssum


def _ln_relu_pair(z, m_ref, gm, bt):
    mu = _dot_hi(z, m_ref[...])
    zc = z - mu
    var = _dot_hi(zc * zc, m_ref[...])
    h = zc * _rsqrt_precise(var + 1e-5) * gm + bt
    return jnp.maximum(h, 0.0)


def _mlp_mid_body(h_ref, g_ref, wt_ref, wb_ref, m_ref, b_ref, gm_ref,
                  bt_ref, o_ref):
    hv = h_ref[...]
    gv = g_ref[...]
    z = _dot(hv, wt_ref[...]) + _dot(gv, wb_ref[...]) + b_ref[...]
    h = _ln_relu_pair(z, m_ref, gm_ref[...], bt_ref[...])
    o_ref[...] = h


def _mlp_last_body(h_ref, g_ref, wt_ref, wb_ref, m_ref, b_ref, gm_ref,
                   bt_ref, o_ref, ss_ref):
    i = pl.program_id(0)
    hv = h_ref[...]
    gv = g_ref[...]
    z = _dot(hv, wt_ref[...]) + _dot(gv, wb_ref[...]) + b_ref[...]
    h = _ln_relu_pair(z, m_ref, gm_ref[...], bt_ref[...])
    o_ref[...] = h

    @pl.when(i == 0)
    def _():
        ss_ref[...] = jnp.zeros_like(ss_ref)
    ss_ref[...] += jnp.sum(h * h, axis=0, keepdims=True)


def _colsq_body(g_ref, ss_ref):
    i = pl.program_id(0)

    @pl.when(i == 0)
    def _():
        ss_ref[...] = jnp.zeros_like(ss_ref)
    g = g_ref[...]
    ss_ref[...] += jnp.sum(g * g, axis=0, keepdims=True)


def _final_body(h_ref, g_ref, ssh_ref, ssg_ref, y_ref):
    ssh_p = ssh_ref[...]
    ssg_p = ssg_ref[...]
    ih = _rsqrt_precise(ssh_p[:, :64] + ssh_p[:, 64:] + 1e-30)
    ig = _rsqrt_precise(ssg_p[:, :64] + ssg_p[:, 64:] + 1e-30)
    hp = h_ref[...]
    gp = g_ref[...]
    yp = jnp.concatenate(
        [hp[:, :64] * ih, gp[:, :64] * ig, hp[:, 64:] * ih, gp[:, 64:] * ig],
        axis=1)
    y_ref[...] = yp.reshape(R, 128)


NP64 = NP * 64
NPH = NP // 2


def _flat_spec():
    return pl.BlockSpec((R // 2, 128), lambda i: (i, 0))


def _const_spec(shape):
    nd = len(shape)
    return pl.BlockSpec(shape, lambda i: (0,) * nd)


def _tc_mlp0(x, cat3, w, b, gm, bt):
    return pl.pallas_call(
        _mlp0_body,
        grid=(GRID,),
        in_specs=[
            pl.BlockSpec((R, 128), lambda i: (i, 0)),
            pl.BlockSpec((1, R, 1), lambda i: (i, 0, 0)),
            _const_spec((128, 64)),
            _const_spec((1, 64)),
            _const_spec((1, 64)),
            _const_spec((1, 64)),
        ],
        out_specs=[pl.BlockSpec((R, 64), lambda i: (i, 0)),
                   _const_spec((1, 128))],
        out_shape=[
            jax.ShapeDtypeStruct((NP, 64), _F32),
            jax.ShapeDtypeStruct((1, 128), _F32),
        ],
    )(x, cat3, w, b, gm, bt)


def _tc_mid(h, g, wt, wb, m, b, gm, bt):
    return pl.pallas_call(
        _mlp_mid_body,
        grid=(GRID,),
        in_specs=[
            _flat_spec(), _flat_spec(),
            _const_spec((128, 128)), _const_spec((128, 128)),
            _const_spec((128, 128)),
            _const_spec((1, 128)), _const_spec((1, 128)),
            _const_spec((1, 128)),
        ],
        out_specs=_flat_spec(),
        out_shape=jax.ShapeDtypeStruct((NPH, 128), _F32),
    )(h, g, wt, wb, m, b, gm, bt)


def _tc_last(h, g, wt, wb, m, b, gm, bt):
    return pl.pallas_call(
        _mlp_last_body,
        grid=(GRID,),
        in_specs=[
            _flat_spec(), _flat_spec(),
            _const_spec((128, 128)), _const_spec((128, 128)),
            _const_spec((128, 128)),
            _const_spec((1, 128)), _const_spec((1, 128)),
            _const_spec((1, 128)),
        ],
        out_specs=[_flat_spec(), _const_spec((1, 128))],
        out_shape=[
            jax.ShapeDtypeStruct((NPH, 128), _F32),
            jax.ShapeDtypeStruct((1, 128), _F32),
        ],
    )(h, g, wt, wb, m, b, gm, bt)


def _tc_colsq(g):
    return pl.pallas_call(
        _colsq_body,
        grid=(GRID,),
        in_specs=[_flat_spec()],
        out_specs=_const_spec((1, 128)),
        out_shape=jax.ShapeDtypeStruct((1, 128), _F32),
    )(g)


def _tc_final(h, g, ssh, ssg):
    return pl.pallas_call(
        _final_body,
        grid=(GRID,),
        in_specs=[
            _flat_spec(), _flat_spec(),
            _const_spec((1, 128)), _const_spec((1, 128)),
        ],
        out_specs=pl.BlockSpec((R, 128), lambda i: (i, 0)),
        out_shape=jax.ShapeDtypeStruct((N, 128), _F32),
    )(h, g, ssh, ssg)


def _sc_mesh():
    return plsc.VectorSubcoreMesh(core_axis_name="c", subcore_axis_name="s",
                                  num_cores=2, num_subcores=16)


def _segmax_gather_body(h_hbm, cat_hbm, st_hbm, g_hbm,
                        sv, cb0, cb1, hb0, hb1, abuf, gout,
                        sem_c0, sem_c1, sem_h0, sem_h1):
    wid = lax.axis_index("s") * 2 + lax.axis_index("c")
    pltpu.sync_copy(st_hbm.at[pl.ds(0, 48)], sv.at[pl.ds(0, 48)])
    sw = sv[pl.ds(wid, 16)]
    start = sw[0]
    end = sw[1]
    c_lo = wid * CW
    c_hi = c_lo + CW

    zv = jnp.zeros((16,), _F32)

    def zb(i, _):
        abuf[pl.ds(i * 16, 16)] = zv
        return 0
    lax.fori_loop(0, (CW + 1) * 4, zb, 0)

    a0 = (start // 8) * 8
    total = end - a0
    nch = (total + CH - 1) // CH
    maxr0 = NP - CH

    def c_off(k):
        return jnp.minimum(a0 + k * CH, maxr0)

    def start_dma(k, cb, hb, sem_c, sem_h):
        r0 = c_off(k)
        pltpu.async_copy(cat_hbm.at[pl.ds(r0, CH)], cb, sem_c)
        pltpu.async_copy(h_hbm.at[pl.ds(r0 * 64, CH * 64)], hb, sem_h)

    def wait_dma(cb, hb, sem_c, sem_h):
        pltpu.make_async_copy(cat_hbm.at[pl.ds(0, CH)], cb, sem_c).wait()
        pltpu.make_async_copy(h_hbm.at[pl.ds(0, CH * 64)], hb, sem_h).wait()

    def scan_chunk(cb, hb, carry):
        def blk(bi, c2):
            prev, q0, q1, q2, q3 = c2
            catv = cb[pl.ds(bi * 16, 16)]
            for j in range(16):
                c = catv[j]
                keep = jnp.where(c != prev, 0.0, 1.0).astype(_F32)
                base = bi * 1024 + j * 64
                q0 = jnp.maximum(hb[pl.ds(base, 16)], q0 * keep)
                q1 = jnp.maximum(hb[pl.ds(base + 16, 16)], q1 * keep)
                q2 = jnp.maximum(hb[pl.ds(base + 32, 16)], q2 * keep)
                q3 = jnp.maximum(hb[pl.ds(base + 48, 16)], q3 * keep)
                lc = jnp.where((c >= c_lo) & (c < c_hi), c - c_lo, CW)
                ab = lc * 64
                abuf[pl.ds(ab, 16)] = q0
                abuf[pl.ds(ab + 16, 16)] = q1
                abuf[pl.ds(ab + 32, 16)] = q2
                abuf[pl.ds(ab + 48, 16)] = q3
                prev = c
            return (prev, q0, q1, q2, q3)
        return lax.fori_loop(0, CH // 16, blk, carry)

    start_dma(0, cb0, hb0, sem_c0, sem_h0)
    npair = jnp.maximum((nch + 1) // 2, 1)

    def pair(i, carry):
        k0 = 2 * i
        wait_dma(cb0, hb0, sem_c0, sem_h0)
        start_dma(k0 + 1, cb1, hb1, sem_c1, sem_h1)
        carry = scan_chunk(cb0, hb0, carry)
        wait_dma(cb1, hb1, sem_c1, sem_h1)
        start_dma(k0 + 2, cb0, hb0, sem_c0, sem_h0)
        carry = scan_chunk(cb1, hb1, carry)
        return carry

    lax.fori_loop(0, npair, pair, (jnp.int32(-1), zv, zv, zv, zv))
    wait_dma(cb0, hb0, sem_c0, sem_h0)

    def chunk2(k, _):
        r0 = c_off(k)
        pltpu.sync_copy(cat_hbm.at[pl.ds(r0, CH)], cb0)
        lo = jnp.maximum(start - r0, 0)
        hi = jnp.minimum(end - r0, CH)

        def blk(bi, _):
            catv = cb0[pl.ds(bi * 16, 16)]
            for j in range(16):
                c = catv[j]
                lc = jnp.where((c >= c_lo) & (c < c_hi), c - c_lo, CW)
                ab = lc * 64
                rb = bi * 1024 + j * 64
                gout[pl.ds(rb, 16)] = abuf[pl.ds(ab, 16)]
                gout[pl.ds(rb + 16, 16)] = abuf[pl.ds(ab + 16, 16)]
                gout[pl.ds(rb + 32, 16)] = abuf[pl.ds(ab + 32, 16)]
                gout[pl.ds(rb + 48, 16)] = abuf[pl.ds(ab + 48, 16)]
            return 0

        lax.fori_loop(0, CH // 16, blk, 0)

        ln = jnp.maximum(hi - lo, 0)
        for b in range(9, -1, -1):
            blen = 1 << b
            rowstart = lo + ((ln >> (b + 1)) << (b + 1))

            @pl.when((ln & blen) != 0)
            def _(rowstart=rowstart, blen=blen):
                pltpu.sync_copy(
                    gout.at[pl.ds(rowstart * 64, blen * 64)],
                    g_hbm.at[pl.ds((r0 + rowstart) * 64, blen * 64)])
        return 0

    lax.fori_loop(0, nch, chunk2, 0)


def _sc_segmax_gather(h_flat, cat, starts):
    run = functools.partial(
        pl.kernel,
        out_type=jax.ShapeDtypeStruct((NP * 64,), _F32),
        mesh=_sc_mesh(),
        scratch_types=[
            pltpu.VMEM((64,), jnp.int32),
            pltpu.VMEM((CH,), jnp.int32),
            pltpu.VMEM((CH,), jnp.int32),
            pltpu.VMEM((CH * 64,), _F32),
            pltpu.VMEM((CH * 64,), _F32),
            pltpu.VMEM(((CW + 1) * 64,), _F32),
            pltpu.VMEM((CH * 64,), _F32),
            pltpu.SemaphoreType.DMA,
            pltpu.SemaphoreType.DMA,
            pltpu.SemaphoreType.DMA,
            pltpu.SemaphoreType.DMA,
        ],
    )(_segmax_gather_body)
    return run(h_flat, cat, starts)


def _bd2(w):
    z = jnp.zeros((128, 128), _F32)
    return z.at[:64, :64].set(w).at[64:, 64:].set(w)


def kernel(x, category, W0, b0, g0, beta0, W1, b1, g1, beta1,
           W2, b2, g2, beta2):
    cat = category.astype(jnp.int32)
    cat_pad = jnp.pad(cat, (0, NP - N))
    cat3 = cat.reshape(GRID, R, 1)

    def r2(v):
        return v.reshape(1, 64)

    def p2(v):
        return jnp.tile(v, 2).reshape(1, 128)

    m_avg = _bd2(jnp.full((64, 64), 1.0 / 64.0, _F32))

    h0, starts_f = _tc_mlp0(x, cat3, W0, r2(b0), r2(g0), r2(beta0))
    starts = starts_f.astype(jnp.int32).reshape(128)

    def fl(v):
        return v.reshape(NP64)

    def pr(v):
        return v.reshape(NPH, 128)

    h0f = h0.reshape(NP64)
    g0v = pr(_sc_segmax_gather(h0f, cat_pad, starts))
    h1 = _tc_mid(pr(h0f), g0v, _bd2(W1[:64]), _bd2(W1[64:]), m_avg,
                 p2(b1), p2(g1), p2(beta1))

    g1v = pr(_sc_segmax_gather(fl(h1), cat_pad, starts))
    h2, ssh = _tc_last(h1, g1v, _bd2(W2[:64]), _bd2(W2[64:]), m_avg,
                       p2(b2), p2(g2), p2(beta2))

    ga2 = _sc_segmax_gather(fl(h2), cat_pad, starts)
    ssg = _tc_colsq(pr(ga2))
    return _tc_final(h2, pr(ga2), ssh, ssg)
